# baseline (speedup 1.0000x reference)
import functools

import numpy as np
import jax
import jax.numpy as jnp
from jax import lax
from jax.experimental import pallas as pl
from jax.experimental.pallas import tpu as pltpu
from jax.experimental.pallas import tpu_sc as plsc

_ATOMS = 51
_V_MIN = -10.0
_V_MAX = 10.0

_BATCH = 524288
_TC_BLK = 16384
_TC_ROWS = 0
_SC_ROWS = _BATCH - _TC_ROWS
_NWORKERS = 32
_WROWS = _SC_ROWS // _NWORKERS
_CROWS = 256

_LN2 = 0.6931471805599453


def _projection_matrix():
    atoms = _ATOMS
    delta = np.float32((_V_MAX - _V_MIN) / (atoms - 1))
    supports = np.linspace(_V_MIN, _V_MAX, atoms, dtype=np.float32)
    tz = np.clip(supports, np.float32(_V_MIN), np.float32(_V_MAX))
    b = ((tz - np.float32(_V_MIN)) / delta).astype(np.float32)
    l = np.floor(b).astype(np.int32)
    u = np.ceil(b).astype(np.int32)
    l = np.where((u > 0) & (l == u), l - 1, l)
    u = np.where((l < atoms - 1) & (l == u), u + 1, u)
    wl = (u.astype(np.float32) - b).astype(np.float32)
    wu = (b - l.astype(np.float32)).astype(np.float32)
    cols = np.arange(atoms)
    mc = np.zeros((atoms, atoms), np.float32)
    np.add.at(mc, (l, cols), wl)
    np.add.at(mc, (u, cols), wu)
    return mc


def _matmul_kernel(a_ref, f_ref, m_ref, o_ref):
    i = pl.program_id(0)
    g = jnp.log(f_ref[...] + 1e-16)
    gl = jnp.dot(g, m_ref[...], preferred_element_type=jnp.float32)
    part = jnp.sum(a_ref[...] * gl, keepdims=True)

    @pl.when(i == 0)
    def _init():
        o_ref[...] = jnp.zeros_like(o_ref)

    o_ref[...] += part


def _tc_partial(anchor, feature, mc, rows, blk):
    grid = rows // blk
    return pl.pallas_call(
        _matmul_kernel,
        grid=(grid,),
        in_specs=[
            pl.BlockSpec((blk, _ATOMS), lambda i: (i, 0)),
            pl.BlockSpec((blk, _ATOMS), lambda i: (i, 0)),
            pl.BlockSpec((_ATOMS, _ATOMS), lambda i: (0, 0)),
        ],
        out_specs=pl.BlockSpec((1, 1), lambda i: (0, 0)),
        out_shape=jax.ShapeDtypeStruct((1, 1), jnp.float32),
    )(anchor, feature, mc)


def _sc_log(x):
    xb = plsc.bitcast(x, jnp.int32)
    e = jnp.right_shift(xb, 23) - 127
    mb = jnp.bitwise_or(jnp.bitwise_and(xb, 0x007FFFFF), 0x3F800000)
    m = plsc.bitcast(mb, jnp.float32)
    s = (m - 1.0) / (m + 1.0)
    z = s * s
    p = z * (1.0 / 9.0) + (1.0 / 7.0)
    p = p * z + (1.0 / 5.0)
    p = p * z + (1.0 / 3.0)
    p = p * z + 1.0
    return e.astype(jnp.float32) * _LN2 + (2.0 * s) * p


def _sc_body(a_hbm, f_hbm, o_hbm, abuf, fbuf, accv):
    wid = lax.axis_index("s") * 2 + lax.axis_index("c")
    base = _TC_ROWS + wid * _WROWS
    tail_w = jnp.where(lax.iota(jnp.int32, 16) >= 13, 1.0, 0.0).astype(
        jnp.float32)

    def chunk_body(ci, acc):
        r0 = base + ci * _CROWS
        pltpu.sync_copy(a_hbm.at[pl.ds(r0, _CROWS), :], abuf)
        pltpu.sync_copy(f_hbm.at[pl.ds(r0, _CROWS), :], fbuf)

        def row_body(i, acc2):
            for off, seg_w in ((0, None), (16, None), (32, None),
                               (35, tail_w)):
                av = abuf[i, pl.ds(off, 16)]
                fv = fbuf[i, pl.ds(off, 16)]
                t = av * _sc_log(fv + 1e-16)
                if seg_w is not None:
                    t = t * seg_w
                acc2 = acc2 + t
            return acc2

        return lax.fori_loop(0, _CROWS, row_body, acc)

    acc = lax.fori_loop(0, _WROWS // _CROWS, chunk_body,
                        jnp.zeros((16,), jnp.float32))
    accv[...] = acc
    pltpu.sync_copy(accv, o_hbm.at[wid])


_sc_partial = functools.partial(
    pl.kernel,
    out_type=jax.ShapeDtypeStruct((_NWORKERS, 16), jnp.float32),
    mesh=plsc.VectorSubcoreMesh(core_axis_name="c", subcore_axis_name="s"),
    compiler_params=pltpu.CompilerParams(needs_layout_passes=False),
    scratch_types=[
        pltpu.VMEM((_CROWS, _ATOMS), jnp.float32),
        pltpu.VMEM((_CROWS, _ATOMS), jnp.float32),
        pltpu.VMEM((16,), jnp.float32),
    ],
)(_sc_body)


_LANES = 128
_WIDTH = _ATOMS * _LANES
_NCHUNKS = 8
_FBLK = 128


def _flat_kernel(a_ref, f_ref, c_ref, o_ref):
    i = pl.program_id(0)
    g = jnp.log(f_ref[...] + 1e-16)
    gl = g * c_ref[0:1, :]
    gl += jnp.roll(g, 1, axis=1) * c_ref[1:2, :]
    gl += jnp.roll(g, -1, axis=1) * c_ref[2:3, :]
    part = jnp.sum(a_ref[...] * gl, keepdims=True)

    @pl.when(i == 0)
    def _init():
        o_ref[...] = jnp.zeros_like(o_ref)

    o_ref[...] += part[0:1, 0:1]


def _flat_coefs(mc_np):
    c0 = np.tile(np.diag(mc_np), _LANES)
    cm = np.tile(np.concatenate([[0.0], np.diag(mc_np, 1)]), _LANES)
    cp = np.tile(np.concatenate([np.diag(mc_np, -1), [0.0]]), _LANES)
    coefs = np.zeros((8, _WIDTH), np.float32)
    coefs[0], coefs[1], coefs[2] = c0, cm, cp
    return jnp.asarray(coefs)


def _flat_partial(a2, f2, coefs):
    rows = a2.shape[0]
    grid = rows // _FBLK
    return pl.pallas_call(
        _flat_kernel,
        grid=(grid,),
        in_specs=[
            pl.BlockSpec((_FBLK, _WIDTH), lambda i: (i, 0)),
            pl.BlockSpec((_FBLK, _WIDTH), lambda i: (i, 0)),
            pl.BlockSpec((8, _WIDTH), lambda i: (0, 0)),
        ],
        out_specs=pl.BlockSpec((1, 1), lambda i: (0, 0)),
        out_shape=jax.ShapeDtypeStruct((1, 1), jnp.float32),
    )(a2, f2, coefs)


def kernel(anchor, feature):
    batch, atoms = anchor.shape
    mc_np = _projection_matrix()
    tridiag = np.array_equal(mc_np, np.tril(np.triu(mc_np, -1), 1))
    total_elems = batch * atoms

    if (tridiag and atoms == _ATOMS
            and total_elems % (_NCHUNKS * _FBLK * _WIDTH) == 0):
        coefs = _flat_coefs(mc_np)
        crows = batch // _NCHUNKS
        frows = total_elems // (_NCHUNKS * _WIDTH)
        parts = []
        for c in range(_NCHUNKS):
            a2 = anchor[c * crows:(c + 1) * crows].reshape(frows, _WIDTH)
            f2 = feature[c * crows:(c + 1) * crows].reshape(frows, _WIDTH)
            parts.append(_flat_partial(a2, f2, coefs)[0, 0])
        total = sum(parts)
    else:
        mc = jnp.asarray(mc_np)
        blk = 16384
        while batch % blk:
            blk //= 2
        total = _tc_partial(anchor, feature, mc, batch, blk)[0, 0]

    return -(total / batch)


# speedup vs baseline: 2.1703x; 2.1703x over previous
import functools

import numpy as np
import jax
import jax.numpy as jnp
from jax import lax
from jax.experimental import pallas as pl
from jax.experimental.pallas import tpu as pltpu
from jax.experimental.pallas import tpu_sc as plsc

_ATOMS = 51
_V_MIN = -10.0
_V_MAX = 10.0

_BATCH = 524288
_TC_BLK = 16384
_TC_ROWS = 0
_SC_ROWS = _BATCH - _TC_ROWS
_NWORKERS = 32
_WROWS = _SC_ROWS // _NWORKERS
_CROWS = 256

_LN2 = 0.6931471805599453


def _projection_matrix():
    atoms = _ATOMS
    delta = np.float32((_V_MAX - _V_MIN) / (atoms - 1))
    supports = np.linspace(_V_MIN, _V_MAX, atoms, dtype=np.float32)
    tz = np.clip(supports, np.float32(_V_MIN), np.float32(_V_MAX))
    b = ((tz - np.float32(_V_MIN)) / delta).astype(np.float32)
    l = np.floor(b).astype(np.int32)
    u = np.ceil(b).astype(np.int32)
    l = np.where((u > 0) & (l == u), l - 1, l)
    u = np.where((l < atoms - 1) & (l == u), u + 1, u)
    wl = (u.astype(np.float32) - b).astype(np.float32)
    wu = (b - l.astype(np.float32)).astype(np.float32)
    cols = np.arange(atoms)
    mc = np.zeros((atoms, atoms), np.float32)
    np.add.at(mc, (l, cols), wl)
    np.add.at(mc, (u, cols), wu)
    return mc


def _matmul_kernel(a_ref, f_ref, m_ref, o_ref):
    i = pl.program_id(0)
    g = jnp.log(f_ref[...] + 1e-16)
    gl = jnp.dot(g, m_ref[...], preferred_element_type=jnp.float32)
    part = jnp.sum(a_ref[...] * gl, keepdims=True)

    @pl.when(i == 0)
    def _init():
        o_ref[...] = jnp.zeros_like(o_ref)

    o_ref[...] += part


def _tc_partial(anchor, feature, mc, rows, blk):
    grid = rows // blk
    return pl.pallas_call(
        _matmul_kernel,
        grid=(grid,),
        in_specs=[
            pl.BlockSpec((blk, _ATOMS), lambda i: (i, 0)),
            pl.BlockSpec((blk, _ATOMS), lambda i: (i, 0)),
            pl.BlockSpec((_ATOMS, _ATOMS), lambda i: (0, 0)),
        ],
        out_specs=pl.BlockSpec((1, 1), lambda i: (0, 0)),
        out_shape=jax.ShapeDtypeStruct((1, 1), jnp.float32),
        compiler_params=pltpu.CompilerParams(
            allow_input_fusion=[True, True, False]),
    )(anchor, feature, mc)


def _sc_log(x):
    xb = plsc.bitcast(x, jnp.int32)
    e = jnp.right_shift(xb, 23) - 127
    mb = jnp.bitwise_or(jnp.bitwise_and(xb, 0x007FFFFF), 0x3F800000)
    m = plsc.bitcast(mb, jnp.float32)
    s = (m - 1.0) / (m + 1.0)
    z = s * s
    p = z * (1.0 / 9.0) + (1.0 / 7.0)
    p = p * z + (1.0 / 5.0)
    p = p * z + (1.0 / 3.0)
    p = p * z + 1.0
    return e.astype(jnp.float32) * _LN2 + (2.0 * s) * p


def _sc_body(a_hbm, f_hbm, o_hbm, abuf, fbuf, accv):
    wid = lax.axis_index("s") * 2 + lax.axis_index("c")
    base = _TC_ROWS + wid * _WROWS
    tail_w = jnp.where(lax.iota(jnp.int32, 16) >= 13, 1.0, 0.0).astype(
        jnp.float32)

    def chunk_body(ci, acc):
        r0 = base + ci * _CROWS
        pltpu.sync_copy(a_hbm.at[pl.ds(r0, _CROWS), :], abuf)
        pltpu.sync_copy(f_hbm.at[pl.ds(r0, _CROWS), :], fbuf)

        def row_body(i, acc2):
            for off, seg_w in ((0, None), (16, None), (32, None),
                               (35, tail_w)):
                av = abuf[i, pl.ds(off, 16)]
                fv = fbuf[i, pl.ds(off, 16)]
                t = av * _sc_log(fv + 1e-16)
                if seg_w is not None:
                    t = t * seg_w
                acc2 = acc2 + t
            return acc2

        return lax.fori_loop(0, _CROWS, row_body, acc)

    acc = lax.fori_loop(0, _WROWS // _CROWS, chunk_body,
                        jnp.zeros((16,), jnp.float32))
    accv[...] = acc
    pltpu.sync_copy(accv, o_hbm.at[wid])


_sc_partial = functools.partial(
    pl.kernel,
    out_type=jax.ShapeDtypeStruct((_NWORKERS, 16), jnp.float32),
    mesh=plsc.VectorSubcoreMesh(core_axis_name="c", subcore_axis_name="s"),
    compiler_params=pltpu.CompilerParams(needs_layout_passes=False),
    scratch_types=[
        pltpu.VMEM((_CROWS, _ATOMS), jnp.float32),
        pltpu.VMEM((_CROWS, _ATOMS), jnp.float32),
        pltpu.VMEM((16,), jnp.float32),
    ],
)(_sc_body)


_LANES = 128
_WIDTH = _ATOMS * _LANES
_NCHUNKS = 8
_FBLK = 128


def _flat_kernel(a_ref, f_ref, c_ref, o_ref):
    i = pl.program_id(0)
    g = jnp.log(f_ref[...] + 1e-16)
    gl = g * c_ref[0:1, :]
    gl += jnp.roll(g, 1, axis=1) * c_ref[1:2, :]
    gl += jnp.roll(g, -1, axis=1) * c_ref[2:3, :]
    part = jnp.sum(a_ref[...] * gl, keepdims=True)

    @pl.when(i == 0)
    def _init():
        o_ref[...] = jnp.zeros_like(o_ref)

    o_ref[...] += part[0:1, 0:1]


def _flat_coefs(mc_np):
    c0 = np.tile(np.diag(mc_np), _LANES)
    cm = np.tile(np.concatenate([[0.0], np.diag(mc_np, 1)]), _LANES)
    cp = np.tile(np.concatenate([np.diag(mc_np, -1), [0.0]]), _LANES)
    coefs = np.zeros((8, _WIDTH), np.float32)
    coefs[0], coefs[1], coefs[2] = c0, cm, cp
    return jnp.asarray(coefs)


def _flat_partial(a2, f2, coefs):
    rows = a2.shape[0]
    grid = rows // _FBLK
    return pl.pallas_call(
        _flat_kernel,
        grid=(grid,),
        in_specs=[
            pl.BlockSpec((_FBLK, _WIDTH), lambda i: (i, 0)),
            pl.BlockSpec((_FBLK, _WIDTH), lambda i: (i, 0)),
            pl.BlockSpec((8, _WIDTH), lambda i: (0, 0)),
        ],
        out_specs=pl.BlockSpec((1, 1), lambda i: (0, 0)),
        out_shape=jax.ShapeDtypeStruct((1, 1), jnp.float32),
    )(a2, f2, coefs)


def kernel(anchor, feature):
    batch, atoms = anchor.shape
    mc_np = _projection_matrix()
    tridiag = np.array_equal(mc_np, np.tril(np.triu(mc_np, -1), 1))
    total_elems = batch * atoms

    if False and (tridiag and atoms == _ATOMS
            and total_elems % (_NCHUNKS * _FBLK * _WIDTH) == 0):
        coefs = _flat_coefs(mc_np)
        crows = batch // _NCHUNKS
        frows = total_elems // (_NCHUNKS * _WIDTH)
        parts = []
        for c in range(_NCHUNKS):
            a2 = anchor[c * crows:(c + 1) * crows].reshape(frows, _WIDTH)
            f2 = feature[c * crows:(c + 1) * crows].reshape(frows, _WIDTH)
            parts.append(_flat_partial(a2, f2, coefs)[0, 0])
        total = sum(parts)
    else:
        mc = jnp.asarray(mc_np)
        blk = 16384
        while batch % blk:
            blk //= 2
        total = _tc_partial(anchor, feature, mc, batch, blk)[0, 0]

    return -(total / batch)
